# baseline (speedup 1.0000x reference)
import functools

import jax
import jax.numpy as jnp
from jax import lax
from jax.experimental import pallas as pl
from jax.experimental.pallas import tpu as pltpu
from jax.experimental.pallas import tpu_sc as plsc

B = 4096
H = 50
D = 64
LS = 20
SEGS = ((0, 2), (2, 9), (9, 20))

NC = 2
NS = 16
NW = NC * NS
RPW = B // NW
NBUF = 2


def _sc_pool_kernel(table_hbm, idx_hbm, out_hbm, idx_v, buf0, buf1, acc,
                    sem0, sem1):
    c = lax.axis_index("c")
    s = lax.axis_index("s")
    wid = s * NC + c
    base = wid * RPW

    bufs = (buf0, buf1)
    sems = (sem0, sem1)

    pltpu.sync_copy(idx_hbm.at[wid], idx_v)

    def gather(j, slot):
        return pltpu.make_async_copy(
            table_hbm.at[idx_v.at[j]], bufs[slot], sems[slot])

    gather(0, 0).start()
    gather(1, 1).start()

    def zero_body(r, _):
        z = jnp.zeros((16,), jnp.float32)
        for d in range(4):
            acc[r, pl.ds(d * 16, 16)] = z
        return 0
    lax.fori_loop(0, RPW, zero_body, 0)

    def accum(slot):
        buf = bufs[slot]

        def row_body(r, _):
            for d in range(4):
                plsc.addupdate(acc.at[r, pl.ds(d * 16, 16)],
                               buf[r, pl.ds(d * 16, 16)])
            return 0
        lax.fori_loop(0, RPW, row_body, 0)

    def main_body(jj, _):
        for slot in range(NBUF):
            j = jj * NBUF + slot
            gather(j, slot).wait()
            accum(slot)

            @pl.when(j + NBUF < H)
            def _():
                gather(j + NBUF, slot).start()
        return 0

    lax.fori_loop(0, H // NBUF, main_body, 0)

    pltpu.sync_copy(acc, out_hbm.at[pl.ds(base, RPW)])


@functools.lru_cache(maxsize=None)
def _sc_pool():
    return pl.kernel(
        _sc_pool_kernel,
        out_type=jax.ShapeDtypeStruct((B, D), jnp.float32),
        mesh=plsc.VectorSubcoreMesh(core_axis_name="c", subcore_axis_name="s",
                                    num_cores=NC, num_subcores=NS),
        scratch_types=[
            pltpu.VMEM((H, RPW), jnp.int32),
            pltpu.VMEM((RPW, 128), jnp.float32),
            pltpu.VMEM((RPW, 128), jnp.float32),
            pltpu.VMEM((RPW, D), jnp.float32),
            pltpu.SemaphoreType.DMA,
            pltpu.SemaphoreType.DMA,
        ],
    )


_PAD_BLK = 8192


def _tc_pad_kernel(in_ref, out_ref):
    v = in_ref[...]
    out_ref[...] = jnp.concatenate([v, jnp.zeros_like(v)], axis=1)


def _tc_pad(item_emb):
    n = item_emb.shape[0]
    return pl.pallas_call(
        _tc_pad_kernel,
        grid=(n // _PAD_BLK,),
        in_specs=[pl.BlockSpec((_PAD_BLK, D), lambda i: (i, 0))],
        out_specs=pl.BlockSpec((_PAD_BLK, 2 * D), lambda i: (i, 0)),
        out_shape=jax.ShapeDtypeStruct((n, 2 * D), jnp.float32),
    )(item_emb)


def _tc_head_kernel(us_ref, mask_ref, y_ref, ob_ref, w_ref,
                    logit_ref, loss_ref):
    x_len = jnp.sum(mask_ref[...], axis=1, keepdims=True)
    user_rep = us_ref[...] / x_len
    wu = lax.dot_general(user_rep, w_ref[...], (((1,), (1,)), ((), ())),
                         preferred_element_type=jnp.float32)
    y = y_ref[...]
    ob = ob_ref[...]
    col = lax.broadcasted_iota(jnp.int32, (1, LS), 1)
    loss = jnp.float32(0.0)
    logit = jnp.zeros((B, LS), jnp.float32)
    for (s, e) in SEGS:
        m = (col >= s) & (col < e)
        wc = jnp.where(m, wu * ob, 0.0)
        row_sum = jnp.sum(wc, axis=1)
        row_mask = (row_sum != 0.0).astype(jnp.float32)
        cnt = jnp.sum(row_mask)
        denom = jnp.sum(jnp.where(m, jnp.exp(wc), 0.0), axis=1)
        dot_y = jnp.sum(wc * y, axis=1)
        nll = jnp.sum(row_mask * (jnp.log(denom) - dot_y))
        loss = loss + jnp.where(cnt > 0, nll / cnt, jnp.float32(0.0))
        mx = jnp.max(jnp.where(m, wu, -1e30), axis=1, keepdims=True)
        ex = jnp.where(m, jnp.exp(wu - mx), 0.0)
        sm = ex / jnp.sum(ex, axis=1, keepdims=True)
        logit = jnp.where(m, sm, logit)
    logit_ref[...] = logit
    loss_ref[...] = jnp.broadcast_to(loss, (1, 1))


def _tc_head(user_sum, x_mask, y, ob, W):
    return pl.pallas_call(
        _tc_head_kernel,
        out_shape=(
            jax.ShapeDtypeStruct((B, LS), jnp.float32),
            jax.ShapeDtypeStruct((1, 1), jnp.float32),
        ),
    )(user_sum, x_mask, y, ob, W)


def kernel(x, x_mask, y, ob, item_emb, W):
    idx3 = x.reshape(NW, RPW, H).transpose(0, 2, 1)
    table2 = _tc_pad(item_emb)
    user_sum = _sc_pool()(table2, idx3)
    logit, loss = _tc_head(user_sum, x_mask, y, ob, W)
    return (logit, loss[0, 0])


# speedup vs baseline: 2.1062x; 2.1062x over previous
import functools

import jax
import jax.numpy as jnp
from jax import lax
from jax.experimental import pallas as pl
from jax.experimental.pallas import tpu as pltpu
from jax.experimental.pallas import tpu_sc as plsc

B = 4096
H = 50
D = 64
LS = 20
SEGS = ((0, 2), (2, 9), (9, 20))

NC = 2
NS = 16
NW = NC * NS
RPW = B // NW
NBUF = 2


def _sc_pool_kernel(table_hbm, idx_hbm, out_hbm, idx_v, buf0, buf1, acc,
                    sem0, sem1):
    c = lax.axis_index("c")
    s = lax.axis_index("s")
    wid = s * NC + c
    base = wid * RPW

    bufs = (buf0, buf1)
    sems = (sem0, sem1)

    pltpu.sync_copy(idx_hbm.at[wid], idx_v)

    def gather(j, slot):
        return pltpu.make_async_copy(
            table_hbm.at[idx_v.at[j]], bufs[slot], sems[slot])

    gather(0, 0).start()
    gather(1, 1).start()

    def zero_body(r, _):
        z = jnp.zeros((16,), jnp.float32)
        for d in range(4):
            acc[r, pl.ds(d * 16, 16)] = z
        return 0
    lax.fori_loop(0, RPW, zero_body, 0)

    def accum(slot):
        buf = bufs[slot]

        def row_body(r, _):
            for d in range(4):
                plsc.addupdate(acc.at[r, pl.ds(d * 16, 16)],
                               buf[r, pl.ds(d * 16, 16)])
            return 0
        lax.fori_loop(0, RPW, row_body, 0)

    def main_body(jj, _):
        for slot in range(NBUF):
            j = jj * NBUF + slot
            gather(j, slot).wait()
            accum(slot)

            @pl.when(j + NBUF < H)
            def _():
                gather(j + NBUF, slot).start()
        return 0

    lax.fori_loop(0, H // NBUF, main_body, 0)

    pltpu.sync_copy(acc, out_hbm.at[pl.ds(base, RPW)])


@functools.lru_cache(maxsize=None)
def _sc_pool():
    return pl.kernel(
        _sc_pool_kernel,
        out_type=jax.ShapeDtypeStruct((B, D), jnp.float32),
        mesh=plsc.VectorSubcoreMesh(core_axis_name="c", subcore_axis_name="s",
                                    num_cores=NC, num_subcores=NS),
        scratch_types=[
            pltpu.VMEM((H, RPW), jnp.int32),
            pltpu.VMEM((RPW, 128), jnp.float32),
            pltpu.VMEM((RPW, 128), jnp.float32),
            pltpu.VMEM((RPW, D), jnp.float32),
            pltpu.SemaphoreType.DMA,
            pltpu.SemaphoreType.DMA,
        ],
    )


_PAD_BLK = 8192


def _tc_pad_kernel(in_ref, out_ref):
    vt = in_ref[...].T
    out_ref[...] = jnp.concatenate([vt, jnp.zeros_like(vt)], axis=1)


def _tc_pad(emb_t):
    n = emb_t.shape[1]
    return pl.pallas_call(
        _tc_pad_kernel,
        grid=((n + _PAD_BLK - 1) // _PAD_BLK,),
        in_specs=[pl.BlockSpec((D, _PAD_BLK), lambda i: (0, i))],
        out_specs=pl.BlockSpec((_PAD_BLK, 2 * D), lambda i: (i, 0)),
        out_shape=jax.ShapeDtypeStruct((n, 2 * D), jnp.float32),
    )(emb_t)


def _tc_head_kernel(us_ref, mask_ref, y_ref, ob_ref, w_ref,
                    logit_ref, loss_ref):
    x_len = jnp.sum(mask_ref[...], axis=1, keepdims=True)
    user_rep = us_ref[...] / x_len
    wu = lax.dot_general(user_rep, w_ref[...], (((1,), (1,)), ((), ())),
                         preferred_element_type=jnp.float32)
    y = y_ref[...]
    ob = ob_ref[...]
    col = lax.broadcasted_iota(jnp.int32, (1, LS), 1)
    loss = jnp.float32(0.0)
    logit = jnp.zeros((B, LS), jnp.float32)
    for (s, e) in SEGS:
        m = (col >= s) & (col < e)
        wc = jnp.where(m, wu * ob, 0.0)
        row_sum = jnp.sum(wc, axis=1)
        row_mask = (row_sum != 0.0).astype(jnp.float32)
        cnt = jnp.sum(row_mask)
        denom = jnp.sum(jnp.where(m, jnp.exp(wc), 0.0), axis=1)
        dot_y = jnp.sum(wc * y, axis=1)
        nll = jnp.sum(row_mask * (jnp.log(denom) - dot_y))
        loss = loss + jnp.where(cnt > 0, nll / cnt, jnp.float32(0.0))
        mx = jnp.max(jnp.where(m, wu, -1e30), axis=1, keepdims=True)
        ex = jnp.where(m, jnp.exp(wu - mx), 0.0)
        sm = ex / jnp.sum(ex, axis=1, keepdims=True)
        logit = jnp.where(m, sm, logit)
    logit_ref[...] = logit
    loss_ref[...] = jnp.broadcast_to(loss, (1, 1))


def _tc_head(user_sum, x_mask, y, ob, W):
    return pl.pallas_call(
        _tc_head_kernel,
        out_shape=(
            jax.ShapeDtypeStruct((B, LS), jnp.float32),
            jax.ShapeDtypeStruct((1, 1), jnp.float32),
        ),
    )(user_sum, x_mask, y, ob, W)


def kernel(x, x_mask, y, ob, item_emb, W):
    idx3 = x.reshape(NW, RPW, H).transpose(0, 2, 1)
    table2 = _tc_pad(item_emb.T)
    user_sum = _sc_pool()(table2, idx3)
    logit, loss = _tc_head(user_sum, x_mask, y, ob, W)
    return (logit, loss[0, 0])
